# Initial kernel scaffold; baseline (speedup 1.0000x reference)
import jax
import jax.numpy as jnp
from jax.experimental import pallas as pl


def kernel(x, edge_index, batch, W_in, b_in, W0, b0, W1, b1, W2, b2, Wc1, bc1, Wc2, bc2):
    raise NotImplementedError("write your pallas kernel here")


# speedup vs baseline: 20.5987x; 20.5987x over previous
import functools

import jax
import jax.numpy as jnp
from jax import lax
from jax.experimental import pallas as pl
from jax.experimental.pallas import tpu as pltpu
from jax.experimental.pallas import tpu_sc as plsc

N = 10000
E = 320000
H = 64
G = 64

NC = 2
NS = 16
NW = NC * NS
EPW = E // NW
K = 80
NCH = EPW // K
NPAD = 10240
RPT = NPAD // NS

_PREC = lax.Precision.HIGHEST


def _sc_count(dst_r):
  mesh = plsc.VectorSubcoreMesh(core_axis_name="c", subcore_axis_name="s")

  @functools.partial(
      pl.kernel,
      mesh=mesh,
      out_type=jax.ShapeDtypeStruct((NW, N), jnp.float32),
      compiler_params=pltpu.CompilerParams(needs_layout_passes=False),
      scratch_types=[
          pltpu.VMEM((EPW,), jnp.int32),
          pltpu.VMEM((N,), jnp.float32),
      ],
  )
  def k(dst_hbm, out_hbm, idx_v, acc_v):
    c = lax.axis_index("c")
    s = lax.axis_index("s")
    w = s * NC + c
    pltpu.sync_copy(dst_hbm.at[w], idx_v)
    zeros16 = jnp.zeros((16,), jnp.float32)

    def zbody(i, carry):
      acc_v[pl.ds(i * 16, 16)] = zeros16
      return carry

    lax.fori_loop(0, N // 16, zbody, 0)
    ones16 = jnp.ones((16,), jnp.float32)

    def body(i, carry):
      idx = idx_v[pl.ds(i * 16, 16)]
      plsc.addupdate_scatter(acc_v, [idx], ones16)
      return carry

    lax.fori_loop(0, EPW // 16, body, 0)
    pltpu.sync_copy(acc_v, out_hbm.at[w])

  return k(dst_r)


def _sc_agg(src_r, dst_r, hws, zrows):
  mesh = plsc.VectorSubcoreMesh(core_axis_name="c", subcore_axis_name="s")

  @functools.partial(
      pl.kernel,
      mesh=mesh,
      out_type=jax.ShapeDtypeStruct((NC, NPAD, H), jnp.float32),
      compiler_params=pltpu.CompilerParams(use_tc_tiling_on_sc=False),
      scratch_types=[
          pltpu.VMEM((NCH, K), jnp.int32),
          pltpu.VMEM((NCH, K), jnp.int32),
          pltpu.VMEM((K, H), jnp.float32),
          pltpu.VMEM_SHARED((NPAD, H), jnp.float32),
          pltpu.SemaphoreType.DMA,
      ],
  )
  def k(src_hbm, dst_hbm, hws_hbm, z_hbm, out_hbm, src_v, dst_v, rows_v,
        acc_sh, sem):
    c = lax.axis_index("c")
    s = lax.axis_index("s")
    w = s * NC + c
    pltpu.sync_copy(z_hbm, acc_sh.at[pl.ds(s * RPT, RPT)])
    pltpu.sync_copy(src_hbm.at[w], src_v)
    pltpu.sync_copy(dst_hbm.at[w], dst_v)
    plsc.subcore_barrier()

    def body(j, carry):
      pltpu.async_copy(hws_hbm.at[src_v.at[j]], rows_v, sem).wait()
      pltpu.sync_copy(rows_v, acc_sh.at[dst_v.at[j]], add=True)
      return carry

    lax.fori_loop(0, NCH, body, 0)
    plsc.subcore_barrier()
    pltpu.sync_copy(acc_sh.at[pl.ds(s * RPT, RPT)],
                    out_hbm.at[c, pl.ds(s * RPT, RPT)])

  return k(src_r, dst_r, hws, zrows)


def _tc_pre(x, W_in, b_in, W0, cnts):
  def body(x_ref, wi_ref, bi_ref, w0_ref, cnt_ref, h0_ref, hws0_ref, dinv_ref):
    deg = jnp.sum(cnt_ref[...], axis=0) + 1.0
    dinv = lax.rsqrt(deg)
    h0 = jax.nn.relu(
        jnp.dot(x_ref[...], wi_ref[...], preferred_element_type=jnp.float32,
                precision=_PREC) + bi_ref[...][None, :])
    hws0 = jnp.dot(h0, w0_ref[...], preferred_element_type=jnp.float32,
                   precision=_PREC) * dinv[:, None]
    h0_ref[...] = h0
    hws0_ref[...] = hws0
    dinv_ref[...] = dinv

  return pl.pallas_call(
      body,
      out_shape=(
          jax.ShapeDtypeStruct((N, H), jnp.float32),
          jax.ShapeDtypeStruct((N, H), jnp.float32),
          jax.ShapeDtypeStruct((N,), jnp.float32),
      ),
  )(x, W_in, b_in, W0, cnts)


def _tc_layer(t_parts, hws_prev, b_prev, W_next, dinv, h_res=None):
  with_res = h_res is not None

  def body(*refs):
    if with_res:
      (t_ref, hwsp_ref, b_ref, w_ref, dinv_ref, hres_ref,
       hn_ref, hwsn_ref) = refs
    else:
      t_ref, hwsp_ref, b_ref, w_ref, dinv_ref, hn_ref, hwsn_ref = refs
    dinv = dinv_ref[...]
    t = t_ref[0][:N] + t_ref[1][:N] + hwsp_ref[...]
    a = dinv[:, None] * t + b_ref[...][None, :]
    hn = jax.nn.relu(a)
    if with_res:
      hn = hres_ref[...] + hn
    hwsn = jnp.dot(hn, w_ref[...], preferred_element_type=jnp.float32,
                   precision=_PREC) * dinv[:, None]
    hn_ref[...] = hn
    hwsn_ref[...] = hwsn

  args = [t_parts, hws_prev, b_prev, W_next, dinv]
  if with_res:
    args.append(h_res)
  return pl.pallas_call(
      body,
      out_shape=(
          jax.ShapeDtypeStruct((N, H), jnp.float32),
          jax.ShapeDtypeStruct((N, H), jnp.float32),
      ),
  )(*args)


def _tc_final(t_parts, hws2, b2, h2, dinv, batch, Wc1, bc1, Wc2, bc2):
  def body(t_ref, hws_ref, b_ref, h2_ref, dinv_ref, batch_ref, wc1_ref,
           bc1_ref, wc2_ref, bc2_ref, out_ref):
    dinv = dinv_ref[...]
    t = t_ref[0][:N] + t_ref[1][:N] + hws_ref[...]
    h3 = h2_ref[...] + jax.nn.relu(dinv[:, None] * t + b_ref[...][None, :])
    gid = lax.broadcasted_iota(jnp.int32, (N, G), 1)
    oh = (batch_ref[...][:, None] == gid).astype(jnp.float32)
    sums = lax.dot_general(oh, h3, ((((0,), (0,)), ((), ()))),
                           preferred_element_type=jnp.float32,
                           precision=_PREC)
    counts = jnp.sum(oh, axis=0)
    pooled = sums / jnp.maximum(counts, 1.0)[:, None]
    z = jax.nn.relu(
        jnp.dot(pooled, wc1_ref[...], preferred_element_type=jnp.float32,
                precision=_PREC) + bc1_ref[...][None, :])
    out_ref[...] = jnp.dot(z, wc2_ref[...], preferred_element_type=jnp.float32,
                           precision=_PREC) + bc2_ref[...][None, :]

  return pl.pallas_call(
      body,
      out_shape=jax.ShapeDtypeStruct((G, jnp.shape(Wc2)[1]), jnp.float32),
  )(t_parts, hws2, b2, h2, dinv, batch, Wc1, bc1, Wc2, bc2)


def kernel(x, edge_index, batch, W_in, b_in, W0, b0, W1, b1, W2, b2,
           Wc1, bc1, Wc2, bc2):
  src = edge_index[0]
  dst = edge_index[1]
  src_r = src.reshape(NW, NCH, K)
  dst_r = dst.reshape(NW, NCH, K)
  dst_flat = dst.reshape(NW, EPW)
  zrows = jnp.zeros((RPT, H), jnp.float32)

  cnts = _sc_count(dst_flat)
  h0, hws0, dinv = _tc_pre(x, W_in, b_in, W0, cnts)

  t0 = _sc_agg(src_r, dst_r, hws0, zrows)
  h1, hws1 = _tc_layer(t0, hws0, b0, W1, dinv)

  t1 = _sc_agg(src_r, dst_r, hws1, zrows)
  h2, hws2 = _tc_layer(t1, hws1, b1, W2, dinv, h_res=h1)

  t2 = _sc_agg(src_r, dst_r, hws2, zrows)
  out = _tc_final(t2, hws2, b2, h2, dinv, batch, Wc1, bc1, Wc2, bc2)
  return out


# speedup vs baseline: 33.3181x; 1.6175x over previous
import functools

import jax
import jax.numpy as jnp
from jax import lax
from jax.experimental import pallas as pl
from jax.experimental.pallas import tpu as pltpu
from jax.experimental.pallas import tpu_sc as plsc

N = 10000
E = 320000
H = 64
G = 64

NC = 2
NS = 16
NW = NC * NS
EPW = E // NW
K = 100
NCH = EPW // K
NPAD = 10240
RPT = NPAD // NS

_PREC = lax.Precision.HIGHEST


def _sc_count(dst_r):
  mesh = plsc.VectorSubcoreMesh(core_axis_name="c", subcore_axis_name="s")

  @functools.partial(
      pl.kernel,
      mesh=mesh,
      out_type=jax.ShapeDtypeStruct((NW, N), jnp.float32),
      compiler_params=pltpu.CompilerParams(needs_layout_passes=False),
      scratch_types=[
          pltpu.VMEM((EPW,), jnp.int32),
          pltpu.VMEM((N,), jnp.float32),
      ],
  )
  def k(dst_hbm, out_hbm, idx_v, acc_v):
    c = lax.axis_index("c")
    s = lax.axis_index("s")
    w = s * NC + c
    pltpu.sync_copy(dst_hbm.at[w], idx_v)
    zeros16 = jnp.zeros((16,), jnp.float32)

    def zbody(i, carry):
      acc_v[pl.ds(i * 16, 16)] = zeros16
      return carry

    lax.fori_loop(0, N // 16, zbody, 0)
    ones16 = jnp.ones((16,), jnp.float32)

    def body(i, carry):
      idx = idx_v[pl.ds(i * 16, 16)]
      plsc.addupdate_scatter(acc_v, [idx], ones16)
      return carry

    lax.fori_loop(0, EPW // 16, body, 0)
    pltpu.sync_copy(acc_v, out_hbm.at[w])

  return k(dst_r)


def _sc_agg(src_r, dst_r, hws, zrows):
  mesh = plsc.VectorSubcoreMesh(core_axis_name="c", subcore_axis_name="s")

  @functools.partial(
      pl.kernel,
      mesh=mesh,
      out_type=jax.ShapeDtypeStruct((NC, NPAD, H), jnp.float32),
      compiler_params=pltpu.CompilerParams(use_tc_tiling_on_sc=False),
      scratch_types=[
          pltpu.VMEM((NCH, K), jnp.int32),
          pltpu.VMEM((NCH, K), jnp.int32),
          pltpu.VMEM((K, H), jnp.float32),
          pltpu.VMEM((K, H), jnp.float32),
          pltpu.VMEM_SHARED((NPAD, H), jnp.float32),
          pltpu.SemaphoreType.DMA,
          pltpu.SemaphoreType.DMA,
          pltpu.SemaphoreType.DMA,
      ],
  )
  def k(src_hbm, dst_hbm, hws_hbm, z_hbm, out_hbm, src_v, dst_v, rows0, rows1,
        acc_sh, sem_i, sem_a, sem_b):
    c = lax.axis_index("c")
    s = lax.axis_index("s")
    w = s * NC + c
    d0 = pltpu.async_copy(z_hbm, acc_sh.at[pl.ds(s * RPT, RPT)], sem_i)
    d1 = pltpu.async_copy(src_hbm.at[w], src_v, sem_i)
    d2 = pltpu.async_copy(dst_hbm.at[w], dst_v, sem_i)
    d0.wait()
    d1.wait()
    d2.wait()
    plsc.subcore_barrier()

    pltpu.async_copy(hws_hbm.at[src_v.at[0]], rows0, sem_a)

    def body(jj, carry):
      j0 = 2 * jj
      g1 = pltpu.async_copy(hws_hbm.at[src_v.at[j0 + 1]], rows1, sem_b)
      pltpu.make_async_copy(hws_hbm.at[src_v.at[j0]], rows0, sem_a).wait()
      pltpu.sync_copy(rows0, acc_sh.at[dst_v.at[j0]], add=True)
      nxt = lax.rem(j0 + 2, NCH)
      pltpu.async_copy(hws_hbm.at[src_v.at[nxt]], rows0, sem_a)
      g1.wait()
      pltpu.sync_copy(rows1, acc_sh.at[dst_v.at[j0 + 1]], add=True)
      return carry

    lax.fori_loop(0, NCH // 2, body, 0)
    pltpu.make_async_copy(hws_hbm.at[src_v.at[0]], rows0, sem_a).wait()
    plsc.subcore_barrier()
    pltpu.sync_copy(acc_sh.at[pl.ds(s * RPT, RPT)],
                    out_hbm.at[c, pl.ds(s * RPT, RPT)])

  return k(src_r, dst_r, hws, zrows)


def _tc_pre(x, W_in, b_in, W0, cnts):
  def body(x_ref, wi_ref, bi_ref, w0_ref, cnt_ref, h0_ref, hws0_ref, dinv_ref):
    deg = jnp.sum(cnt_ref[...], axis=0) + 1.0
    dinv = lax.rsqrt(deg)
    h0 = jax.nn.relu(
        jnp.dot(x_ref[...], wi_ref[...], preferred_element_type=jnp.float32,
                precision=_PREC) + bi_ref[...][None, :])
    hws0 = jnp.dot(h0, w0_ref[...], preferred_element_type=jnp.float32,
                   precision=_PREC) * dinv[:, None]
    h0_ref[...] = h0
    hws0_ref[...] = hws0
    dinv_ref[...] = dinv

  return pl.pallas_call(
      body,
      out_shape=(
          jax.ShapeDtypeStruct((N, H), jnp.float32),
          jax.ShapeDtypeStruct((N, H), jnp.float32),
          jax.ShapeDtypeStruct((N,), jnp.float32),
      ),
  )(x, W_in, b_in, W0, cnts)


def _tc_layer(t_parts, hws_prev, b_prev, W_next, dinv, h_res=None):
  with_res = h_res is not None

  def body(*refs):
    if with_res:
      (t_ref, hwsp_ref, b_ref, w_ref, dinv_ref, hres_ref,
       hn_ref, hwsn_ref) = refs
    else:
      t_ref, hwsp_ref, b_ref, w_ref, dinv_ref, hn_ref, hwsn_ref = refs
    dinv = dinv_ref[...]
    t = t_ref[0][:N] + t_ref[1][:N] + hwsp_ref[...]
    a = dinv[:, None] * t + b_ref[...][None, :]
    hn = jax.nn.relu(a)
    if with_res:
      hn = hres_ref[...] + hn
    hwsn = jnp.dot(hn, w_ref[...], preferred_element_type=jnp.float32,
                   precision=_PREC) * dinv[:, None]
    hn_ref[...] = hn
    hwsn_ref[...] = hwsn

  args = [t_parts, hws_prev, b_prev, W_next, dinv]
  if with_res:
    args.append(h_res)
  return pl.pallas_call(
      body,
      out_shape=(
          jax.ShapeDtypeStruct((N, H), jnp.float32),
          jax.ShapeDtypeStruct((N, H), jnp.float32),
      ),
  )(*args)


def _tc_final(t_parts, hws2, b2, h2, dinv, batch, Wc1, bc1, Wc2, bc2):
  def body(t_ref, hws_ref, b_ref, h2_ref, dinv_ref, batch_ref, wc1_ref,
           bc1_ref, wc2_ref, bc2_ref, out_ref):
    dinv = dinv_ref[...]
    t = t_ref[0][:N] + t_ref[1][:N] + hws_ref[...]
    h3 = h2_ref[...] + jax.nn.relu(dinv[:, None] * t + b_ref[...][None, :])
    gid = lax.broadcasted_iota(jnp.int32, (N, G), 1)
    oh = (batch_ref[...][:, None] == gid).astype(jnp.float32)
    sums = lax.dot_general(oh, h3, ((((0,), (0,)), ((), ()))),
                           preferred_element_type=jnp.float32,
                           precision=_PREC)
    counts = jnp.sum(oh, axis=0)
    pooled = sums / jnp.maximum(counts, 1.0)[:, None]
    z = jax.nn.relu(
        jnp.dot(pooled, wc1_ref[...], preferred_element_type=jnp.float32,
                precision=_PREC) + bc1_ref[...][None, :])
    out_ref[...] = jnp.dot(z, wc2_ref[...], preferred_element_type=jnp.float32,
                           precision=_PREC) + bc2_ref[...][None, :]

  return pl.pallas_call(
      body,
      out_shape=jax.ShapeDtypeStruct((G, jnp.shape(Wc2)[1]), jnp.float32),
  )(t_parts, hws2, b2, h2, dinv, batch, Wc1, bc1, Wc2, bc2)


def kernel(x, edge_index, batch, W_in, b_in, W0, b0, W1, b1, W2, b2,
           Wc1, bc1, Wc2, bc2):
  src = edge_index[0]
  dst = edge_index[1]
  src_r = src.reshape(NW, NCH, K)
  dst_r = dst.reshape(NW, NCH, K)
  dst_flat = dst.reshape(NW, EPW)
  zrows = jnp.zeros((RPT, H), jnp.float32)

  cnts = _sc_count(dst_flat)
  h0, hws0, dinv = _tc_pre(x, W_in, b_in, W0, cnts)

  t0 = _sc_agg(src_r, dst_r, hws0, zrows)
  h1, hws1 = _tc_layer(t0, hws0, b0, W1, dinv)

  t1 = _sc_agg(src_r, dst_r, hws1, zrows)
  h2, hws2 = _tc_layer(t1, hws1, b1, W2, dinv, h_res=h1)

  t2 = _sc_agg(src_r, dst_r, hws2, zrows)
  out = _tc_final(t2, hws2, b2, h2, dinv, batch, Wc1, bc1, Wc2, bc2)
  return out
